# baseline (speedup 1.0000x reference)
import jax
import jax.numpy as jnp
from jax.experimental import pallas as pl
from jax.experimental.pallas import tpu as pltpu

B, S, H = 1, 2048, 1024
E, K, I = 8, 2, 1024
ALPHA, LIMIT = 1.702, 7.0
T = B * S
TB = 512
NT = T // TB
P = 8
CW = 2 * I // P
CC = CW // 2


def _moe_kernel(x_ref, rw_ref, rb_ref, wgu_ref, bgu_ref,
                wd_ref, bd_ref, sel_ref, out_ref,
                scores_ref, wgu_bf, wd_bf):
    e = pl.program_id(0)
    t = pl.program_id(1)
    rows = pl.ds(t * TB, TB)
    eids = jax.lax.broadcasted_iota(jnp.int32, (TB, E), 1)

    @pl.when(e == 0)
    def _router():
        x = x_ref[rows, :]
        logits = jnp.dot(x, rw_ref[...], preferred_element_type=jnp.float32)
        logits = logits + rb_ref[...]
        v0 = jnp.max(logits, axis=-1, keepdims=True)
        cand0 = jnp.where(logits == v0, eids, E)
        i0 = jnp.min(cand0, axis=-1, keepdims=True)
        masked = jnp.where(eids == i0, -jnp.inf, logits)
        v1 = jnp.max(masked, axis=-1, keepdims=True)
        cand1 = jnp.where(masked == v1, eids, E)
        i1 = jnp.min(cand1, axis=-1, keepdims=True)
        w0 = 1.0 / (1.0 + jnp.exp(v1 - v0))
        w1 = 1.0 - w0
        scores_ref[rows, :] = w0 * (eids == i0) + w1 * (eids == i1)

    @pl.when(t == 0)
    def _cast_weights():
        wgu_bf[...] = wgu_ref[0].astype(jnp.bfloat16)
        wd_bf[...] = wd_ref[0].astype(jnp.bfloat16)

    sc = jnp.sum(scores_ref[rows, :] * (eids == e), axis=-1, keepdims=True)

    x = x_ref[rows, :].astype(jnp.bfloat16)
    gu = jnp.dot(x, wgu_bf[...], preferred_element_type=jnp.float32)
    gu = gu + bgu_ref[0]
    parts = []
    for p in range(P):
        guc = gu[:, p * CW:(p + 1) * CW]
        g = jnp.minimum(guc, LIMIT)
        glu = g * jax.nn.sigmoid(g * ALPHA)
        u = jnp.clip(guc, -LIMIT, LIMIT) + 1.0
        u_shift = pltpu.roll(u, CW - 1, 1)
        pair = (glu * u_shift).astype(jnp.bfloat16)
        parts.append(jnp.dot(pair, sel_ref[...],
                             preferred_element_type=jnp.float32))
    act = jnp.concatenate(parts, axis=1).astype(jnp.bfloat16)
    y = jnp.dot(act, wd_bf[...], preferred_element_type=jnp.float32)
    y = y + bd_ref[0]
    contrib = sc * y

    @pl.when(e == 0)
    def _init():
        out_ref[rows, :] = contrib

    @pl.when(e > 0)
    def _acc():
        out_ref[rows, :] += contrib


@jax.jit
def kernel(hidden_states, router_weight, router_bias, gate_up_proj,
           gate_up_bias, down_proj, down_bias):
    b, s, h = hidden_states.shape
    x = hidden_states.reshape(-1, h)

    b_gu = gate_up_bias.reshape(E, 1, 2 * I)
    b_d = down_bias.reshape(E, 1, H)
    rr = jax.lax.broadcasted_iota(jnp.int32, (CW, CC), 0)
    cc = jax.lax.broadcasted_iota(jnp.int32, (CW, CC), 1)
    sel = (rr == 2 * cc).astype(jnp.bfloat16)

    out = pl.pallas_call(
        _moe_kernel,
        grid=(E, NT),
        in_specs=[
            pl.BlockSpec((T, H), lambda e, t: (0, 0)),
            pl.BlockSpec((H, E), lambda e, t: (0, 0)),
            pl.BlockSpec((E,), lambda e, t: (0,)),
            pl.BlockSpec((1, H, 2 * I), lambda e, t: (e, 0, 0)),
            pl.BlockSpec((1, 1, 2 * I), lambda e, t: (e, 0, 0)),
            pl.BlockSpec((1, I, H), lambda e, t: (e, 0, 0)),
            pl.BlockSpec((1, 1, H), lambda e, t: (e, 0, 0)),
            pl.BlockSpec((CW, CC), lambda e, t: (0, 0)),
        ],
        out_specs=pl.BlockSpec((T, H), lambda e, t: (0, 0)),
        out_shape=jax.ShapeDtypeStruct((T, H), jnp.float32),
        scratch_shapes=[
            pltpu.VMEM((T, E), jnp.float32),
            pltpu.VMEM((H, 2 * I), jnp.bfloat16),
            pltpu.VMEM((I, H), jnp.bfloat16),
        ],
        compiler_params=pltpu.CompilerParams(
            dimension_semantics=("arbitrary", "arbitrary"),
        ),
    )(x, router_weight, router_bias, gate_up_proj, b_gu, down_proj, b_d, sel)

    return out.reshape(b, s, h)


# speedup vs baseline: 1.1140x; 1.1140x over previous
import jax
import jax.numpy as jnp
from jax.experimental import pallas as pl
from jax.experimental.pallas import tpu as pltpu

B, S, H = 1, 2048, 1024
E, K, I = 8, 2, 1024
ALPHA, LIMIT = 1.702, 7.0
T = B * S
TB = 512
NT = T // TB
P = 8
CW = 2 * I // P
CC = CW // 2


def _moe_kernel(x_ref, rw_ref, rb_ref, wgu_ref, bgu_ref,
                wd_ref, bd_ref, sel_ref, out_ref,
                scores_ref, xbf_ref, wgu_bf, wd_bf):
    e = pl.program_id(0)
    t = pl.program_id(1)
    rows = pl.ds(t * TB, TB)
    eids = jax.lax.broadcasted_iota(jnp.int32, (TB, E), 1)

    @pl.when(e == 0)
    def _router():
        x = x_ref[rows, :]
        xbf_ref[rows, :] = x.astype(jnp.bfloat16)
        logits = jnp.dot(x, rw_ref[...], preferred_element_type=jnp.float32)
        logits = logits + rb_ref[...]
        v0 = jnp.max(logits, axis=-1, keepdims=True)
        cand0 = jnp.where(logits == v0, eids, E)
        i0 = jnp.min(cand0, axis=-1, keepdims=True)
        masked = jnp.where(eids == i0, -jnp.inf, logits)
        v1 = jnp.max(masked, axis=-1, keepdims=True)
        cand1 = jnp.where(masked == v1, eids, E)
        i1 = jnp.min(cand1, axis=-1, keepdims=True)
        w0 = 1.0 / (1.0 + jnp.exp(v1 - v0))
        w1 = 1.0 - w0
        scores_ref[rows, :] = w0 * (eids == i0) + w1 * (eids == i1)

    @pl.when(t == 0)
    def _cast_weights():
        wgu_bf[...] = wgu_ref[0].astype(jnp.bfloat16)
        wd_bf[...] = wd_ref[0].astype(jnp.bfloat16)

    sc = jnp.sum(scores_ref[rows, :] * (eids == e), axis=-1, keepdims=True)

    x = xbf_ref[rows, :]
    gu = jnp.dot(x, wgu_bf[...], preferred_element_type=jnp.float32)
    gu = (gu + bgu_ref[0]).astype(jnp.bfloat16)
    parts = []
    for p in range(P):
        guc = gu[:, p * CW:(p + 1) * CW]
        gcuc = jnp.dot(guc, sel_ref[...], preferred_element_type=jnp.float32)
        g = jnp.minimum(gcuc[:, :CC], LIMIT)
        u = jnp.clip(gcuc[:, CC:], -LIMIT, LIMIT)
        glu = g * jax.nn.sigmoid(g * ALPHA)
        parts.append(((u + 1.0) * glu).astype(jnp.bfloat16))
    act = jnp.concatenate(parts, axis=1)
    y = jnp.dot(act, wd_bf[...], preferred_element_type=jnp.float32)
    y = y + bd_ref[0]
    contrib = sc * y

    @pl.when(e == 0)
    def _init():
        out_ref[rows, :] = contrib

    @pl.when(e > 0)
    def _acc():
        out_ref[rows, :] += contrib


@jax.jit
def kernel(hidden_states, router_weight, router_bias, gate_up_proj,
           gate_up_bias, down_proj, down_bias):
    b, s, h = hidden_states.shape
    x = hidden_states.reshape(-1, h)

    b_gu = gate_up_bias.reshape(E, 1, 2 * I)
    b_d = down_bias.reshape(E, 1, H)
    rr = jax.lax.broadcasted_iota(jnp.int32, (CW, CW), 0)
    cc = jax.lax.broadcasted_iota(jnp.int32, (CW, CW), 1)
    sel = ((rr == 2 * cc) | (rr == 2 * (cc - CC) + 1)).astype(jnp.bfloat16)

    out = pl.pallas_call(
        _moe_kernel,
        grid=(E, NT),
        in_specs=[
            pl.BlockSpec((T, H), lambda e, t: (0, 0)),
            pl.BlockSpec((H, E), lambda e, t: (0, 0)),
            pl.BlockSpec((E,), lambda e, t: (0,)),
            pl.BlockSpec((1, H, 2 * I), lambda e, t: (e, 0, 0)),
            pl.BlockSpec((1, 1, 2 * I), lambda e, t: (e, 0, 0)),
            pl.BlockSpec((1, I, H), lambda e, t: (e, 0, 0)),
            pl.BlockSpec((1, 1, H), lambda e, t: (e, 0, 0)),
            pl.BlockSpec((CW, CW), lambda e, t: (0, 0)),
        ],
        out_specs=pl.BlockSpec((T, H), lambda e, t: (0, 0)),
        out_shape=jax.ShapeDtypeStruct((T, H), jnp.float32),
        scratch_shapes=[
            pltpu.VMEM((T, E), jnp.float32),
            pltpu.VMEM((T, H), jnp.bfloat16),
            pltpu.VMEM((H, 2 * I), jnp.bfloat16),
            pltpu.VMEM((I, H), jnp.bfloat16),
        ],
        compiler_params=pltpu.CompilerParams(
            dimension_semantics=("arbitrary", "arbitrary"),
        ),
    )(x, router_weight, router_bias, gate_up_proj, b_gu, down_proj, b_d, sel)

    return out.reshape(b, s, h)


# speedup vs baseline: 1.1494x; 1.0318x over previous
import jax
import jax.numpy as jnp
from jax.experimental import pallas as pl
from jax.experimental.pallas import tpu as pltpu

B, S, H = 1, 2048, 1024
E, K, I = 8, 2, 1024
ALPHA, LIMIT = 1.702, 7.0
T = B * S
TB = 1024
NT = T // TB
P = 8
CW = 2 * I // P
CC = CW // 2


def _moe_kernel(x_ref, rw_ref, rb_ref, wgu_ref, bgu_ref,
                wd_ref, bd_ref, sel_ref, out_ref,
                scores_ref, wgu_bf, wd_bf):
    e = pl.program_id(0)
    t = pl.program_id(1)
    rows = pl.ds(t * TB, TB)
    eids = jax.lax.broadcasted_iota(jnp.int32, (TB, E), 1)

    @pl.when(e == 0)
    def _router():
        x = x_ref[rows, :]
        logits = jnp.dot(x, rw_ref[...], preferred_element_type=jnp.float32)
        logits = logits + rb_ref[...]
        v0 = jnp.max(logits, axis=-1, keepdims=True)
        cand0 = jnp.where(logits == v0, eids, E)
        i0 = jnp.min(cand0, axis=-1, keepdims=True)
        masked = jnp.where(eids == i0, -jnp.inf, logits)
        v1 = jnp.max(masked, axis=-1, keepdims=True)
        cand1 = jnp.where(masked == v1, eids, E)
        i1 = jnp.min(cand1, axis=-1, keepdims=True)
        w0 = 1.0 / (1.0 + jnp.exp(v1 - v0))
        w1 = 1.0 - w0
        scores_ref[rows, :] = w0 * (eids == i0) + w1 * (eids == i1)

    @pl.when(t == 0)
    def _cast_weights():
        wgu_bf[...] = wgu_ref[0].astype(jnp.bfloat16)
        wd_bf[...] = wd_ref[0].astype(jnp.bfloat16)

    sc = jnp.sum(scores_ref[rows, :] * (eids == e), axis=-1, keepdims=True)

    x = x_ref[rows, :].astype(jnp.bfloat16)
    gu = jnp.dot(x, wgu_bf[...], preferred_element_type=jnp.float32)
    gu = (gu + bgu_ref[0]).astype(jnp.bfloat16)
    parts = []
    for p in range(P):
        guc = gu[:, p * CW:(p + 1) * CW]
        gcuc = jnp.dot(guc, sel_ref[...], preferred_element_type=jnp.float32)
        g = jnp.minimum(gcuc[:, :CC], LIMIT)
        u = jnp.clip(gcuc[:, CC:], -LIMIT, LIMIT)
        glu = g * jax.nn.sigmoid(g * ALPHA)
        parts.append(((u + 1.0) * glu).astype(jnp.bfloat16))
    act = jnp.concatenate(parts, axis=1)
    y = jnp.dot(act, wd_bf[...], preferred_element_type=jnp.float32)
    y = y + bd_ref[0]
    contrib = sc * y

    @pl.when(e == 0)
    def _init():
        out_ref[rows, :] = contrib

    @pl.when(e > 0)
    def _acc():
        out_ref[rows, :] += contrib


@jax.jit
def kernel(hidden_states, router_weight, router_bias, gate_up_proj,
           gate_up_bias, down_proj, down_bias):
    b, s, h = hidden_states.shape
    x = hidden_states.reshape(-1, h)

    b_gu = gate_up_bias.reshape(E, 1, 2 * I)
    b_d = down_bias.reshape(E, 1, H)
    rr = jax.lax.broadcasted_iota(jnp.int32, (CW, CW), 0)
    cc = jax.lax.broadcasted_iota(jnp.int32, (CW, CW), 1)
    sel = ((rr == 2 * cc) | (rr == 2 * (cc - CC) + 1)).astype(jnp.bfloat16)

    out = pl.pallas_call(
        _moe_kernel,
        grid=(E, NT),
        in_specs=[
            pl.BlockSpec((T, H), lambda e, t: (0, 0)),
            pl.BlockSpec((H, E), lambda e, t: (0, 0)),
            pl.BlockSpec((E,), lambda e, t: (0,)),
            pl.BlockSpec((1, H, 2 * I), lambda e, t: (e, 0, 0)),
            pl.BlockSpec((1, 1, 2 * I), lambda e, t: (e, 0, 0)),
            pl.BlockSpec((1, I, H), lambda e, t: (e, 0, 0)),
            pl.BlockSpec((1, 1, H), lambda e, t: (e, 0, 0)),
            pl.BlockSpec((CW, CW), lambda e, t: (0, 0)),
        ],
        out_specs=pl.BlockSpec((T, H), lambda e, t: (0, 0)),
        out_shape=jax.ShapeDtypeStruct((T, H), jnp.float32),
        scratch_shapes=[
            pltpu.VMEM((T, E), jnp.float32),
            pltpu.VMEM((H, 2 * I), jnp.bfloat16),
            pltpu.VMEM((I, H), jnp.bfloat16),
        ],
        compiler_params=pltpu.CompilerParams(
            dimension_semantics=("arbitrary", "arbitrary"),
        ),
    )(x, router_weight, router_bias, gate_up_proj, b_gu, down_proj, b_d, sel)

    return out.reshape(b, s, h)
